# baseline (speedup 1.0000x reference)
import functools

import jax
import jax.numpy as jnp
from jax import lax
from jax.experimental import pallas as pl
from jax.experimental.pallas import tpu as pltpu
from jax.experimental.pallas import tpu_sc as plsc

_N = 50000
_E = 800000
_G = 64
_B = 1000
_NB = _N // _B

_RPB = 3
_NBLK = 68
_RPW = _RPB * _NBLK
_NR = 32 * _RPW
_EPAD = 128 * _NR
_TPR = 3128
_NACC = 16 * _TPR


def _ln_relu(t, g, b):
    m = jnp.mean(t, axis=-1, keepdims=True)
    d = t - m
    v = jnp.mean(d * d, axis=-1, keepdims=True)
    y = d * lax.rsqrt(v + 1e-5) * g + b
    return jnp.maximum(y, 0.0)


def _lin_body(x_ref, w_ref, o_ref):
    o_ref[...] = jnp.dot(x_ref[...], w_ref[...],
                         preferred_element_type=jnp.float32)


def _linear(x, wT):
    din, dout = wT.shape
    return pl.pallas_call(
        _lin_body,
        grid=(_NB,),
        in_specs=[
            pl.BlockSpec((_B, din), lambda i: (i, 0)),
            pl.BlockSpec((din, dout), lambda i: (0, 0)),
        ],
        out_specs=pl.BlockSpec((_B, dout), lambda i: (i, 0)),
        out_shape=jax.ShapeDtypeStruct((_N, dout), jnp.float32),
    )(x, wT)


def _make_layer(din, dout, dh, with_wr):
    def body(a_ref, h_ref, wrT_ref, br_ref, wtT_ref, g_ref, b_ref, o_ref):
        a = a_ref[0] + a_ref[1]
        if with_wr:
            t = jnp.dot(a, wrT_ref[...], preferred_element_type=jnp.float32)
        else:
            t = a
        t += br_ref[...]
        t += jnp.dot(h_ref[...], wtT_ref[...],
                     preferred_element_type=jnp.float32)
        o_ref[...] = _ln_relu(t, g_ref[...], b_ref[...])

    def run(aggr, h, wrT, br, wtT, g, b):
        return pl.pallas_call(
            body,
            grid=(_NB,),
            in_specs=[
                pl.BlockSpec((2, _B, din), lambda i: (0, i, 0)),
                pl.BlockSpec((_B, dh), lambda i: (i, 0)),
                pl.BlockSpec(wrT.shape, lambda i: (0, 0)),
                pl.BlockSpec((1, dout), lambda i: (0, 0)),
                pl.BlockSpec((dh, dout), lambda i: (0, 0)),
                pl.BlockSpec((1, dout), lambda i: (0, 0)),
                pl.BlockSpec((1, dout), lambda i: (0, 0)),
            ],
            out_specs=pl.BlockSpec((_B, dout), lambda i: (i, 0)),
            out_shape=jax.ShapeDtypeStruct((_N, dout), jnp.float32),
        )(aggr, h, wrT, br, wtT, g, b)

    return run


def _layer4_pool_body(a_ref, h_ref, wrT_ref, br_ref, wtT_ref, g_ref, b_ref,
                      batch_ref, o_ref, sums, cnt):
    i = pl.program_id(0)

    @pl.when(i == 0)
    def _init():
        sums[...] = jnp.zeros_like(sums)
        cnt[...] = jnp.zeros_like(cnt)

    t = jnp.dot(a_ref[0] + a_ref[1], wrT_ref[...],
                preferred_element_type=jnp.float32)
    t += br_ref[...]
    t += jnp.dot(h_ref[...], wtT_ref[...], preferred_element_type=jnp.float32)
    t = _ln_relu(t, g_ref[...], b_ref[...])

    bb = batch_ref[0]
    mask = (lax.broadcasted_iota(jnp.int32, (_G, _B), 0) == bb).astype(jnp.float32)
    sums[...] += jnp.dot(mask, t, preferred_element_type=jnp.float32)
    cnt[...] += jnp.sum(mask, axis=1, keepdims=True)

    @pl.when(i == _NB - 1)
    def _fin():
        o_ref[...] = sums[...] / jnp.maximum(cnt[:, 0:1], 1.0)


def _layer4_pool(aggr, h, wrT, br, wtT, g, b, batch3):
    din, dout = wrT.shape
    return pl.pallas_call(
        _layer4_pool_body,
        grid=(_NB,),
        in_specs=[
            pl.BlockSpec((2, _B, din), lambda i: (0, i, 0)),
            pl.BlockSpec((_B, din), lambda i: (i, 0)),
            pl.BlockSpec((din, dout), lambda i: (0, 0)),
            pl.BlockSpec((1, dout), lambda i: (0, 0)),
            pl.BlockSpec((din, dout), lambda i: (0, 0)),
            pl.BlockSpec((1, dout), lambda i: (0, 0)),
            pl.BlockSpec((1, dout), lambda i: (0, 0)),
            pl.BlockSpec((1, 1, _B), lambda i: (i, 0, 0)),
        ],
        out_specs=pl.BlockSpec((_G, dout), lambda i: (0, 0)),
        out_shape=jax.ShapeDtypeStruct((_G, dout), jnp.float32),
        scratch_shapes=[
            pltpu.VMEM((_G, dout), jnp.float32),
            pltpu.VMEM((_G, 128), jnp.float32),
        ],
    )(aggr, h, wrT, br, wtT, g, b, batch3)


def _make_segsum_sc(nf):
    mesh = plsc.VectorSubcoreMesh(core_axis_name="c", subcore_axis_name="s")
    BE = _RPB * 128

    @functools.partial(
        pl.kernel, mesh=mesh,
        compiler_params=pltpu.CompilerParams(use_tc_tiling_on_sc=False),
        out_type=jax.ShapeDtypeStruct((2, _N, nf * 32), jnp.float32),
        scratch_types=[
            pltpu.VMEM_SHARED((_NACC, 32), jnp.float32),
            pltpu.VMEM((BE, 32), jnp.float32),
            pltpu.VMEM((BE, 32), jnp.float32),
            pltpu.VMEM((4, _RPB, 128), jnp.int32),
            pltpu.VMEM((4, _RPB, 128), jnp.int32),
            pltpu.SemaphoreType.DMA,
            pltpu.SemaphoreType.DMA,
            pltpu.SemaphoreType.DMA,
            pltpu.SemaphoreType.DMA,
            pltpu.SemaphoreType.DMA,
            pltpu.SemaphoreType.DMA,
            pltpu.SemaphoreType.DMA,
            pltpu.SemaphoreType.DMA,
        ],
    )
    def seg(ytab, gidx3, dst2, zeros, out, acc, buf0, buf1, ig, idx_d,
            semg0, semg1, sems0, sems1, si0, si1, si2, si3):
        c = lax.axis_index("c")
        s = lax.axis_index("s")
        wr0 = (c * 16 + s) * _RPW
        bufs = (buf0, buf1)
        semg = (semg0, semg1)
        sems = (sems0, sems1)
        semi = (si0, si1, si2, si3)

        for fc in range(nf):

            def fire_idx(bb, slot):
                pltpu.async_copy(gidx3.at[fc, pl.ds(wr0 + bb * _RPB, _RPB)],
                                 ig.at[slot], semi[slot])
                pltpu.async_copy(dst2.at[pl.ds(wr0 + bb * _RPB, _RPB)],
                                 idx_d.at[slot], semi[slot])

            def drain_idx(slot):
                pltpu.make_async_copy(gidx3.at[fc, pl.ds(0, _RPB)],
                                      ig.at[slot], semi[slot]).wait()
                pltpu.make_async_copy(dst2.at[pl.ds(0, _RPB)],
                                      idx_d.at[slot], semi[slot]).wait()

            def fire(slot, par):
                for j in range(_RPB):
                    pltpu.async_copy(ytab.at[ig.at[slot, j]],
                                     bufs[par].at[pl.ds(j * 128, 128)],
                                     semg[par])

            def drain_g(par):
                pltpu.make_async_copy(ytab.at[pl.ds(0, BE)], bufs[par],
                                      semg[par]).wait()

            def scat(slot, par):
                for j in range(_RPB):
                    pltpu.async_copy(bufs[par].at[pl.ds(j * 128, 128)],
                                     acc.at[idx_d.at[slot, j]], sems[par],
                                     add=True)

            def drain_s(par):
                pltpu.make_async_copy(ytab.at[pl.ds(0, BE)], bufs[par],
                                      sems[par]).wait()

            pltpu.sync_copy(zeros.at[pl.ds(s * _TPR, _TPR)],
                            acc.at[pl.ds(s * _TPR, _TPR)])
            plsc.subcore_barrier()

            fire_idx(0, 0)
            fire_idx(1, 1)
            fire_idx(2, 2)
            drain_idx(0)
            fire(0, 0)

            def quad(i, carry):
                for q in range(4):
                    p = q % 2
                    if q == 0:
                        @pl.when(i >= 1)
                        def _():
                            drain_s(1)
                    else:
                        drain_s(1 - p)
                    if q == 0:
                        fire_idx(4 * i + 3, 3)
                    else:
                        @pl.when(i <= 15)
                        def _():
                            fire_idx(4 * i + q + 3, (q + 3) % 4)
                    if q == 3:
                        @pl.when(i <= 15)
                        def _():
                            drain_idx(0)
                            fire(0, 1 - p)
                    else:
                        drain_idx(q + 1)
                        fire(q + 1, 1 - p)
                    drain_g(p)
                    scat(q, p)
                return carry

            lax.fori_loop(0, _NBLK // 4, quad, 0)
            drain_s(1)

            plsc.subcore_barrier()
            @pl.when(s <= 14)
            def _wr():
                pltpu.sync_copy(
                    acc.at[pl.ds(s * _TPR, _TPR)],
                    out.at[c, pl.ds(s * _TPR, _TPR), pl.ds(fc * 32, 32)])

            @pl.when(s == 15)
            def _wr_last():
                pltpu.sync_copy(
                    acc.at[pl.ds(15 * _TPR, _N - 15 * _TPR)],
                    out.at[c, pl.ds(15 * _TPR, _N - 15 * _TPR),
                           pl.ds(fc * 32, 32)])

    return seg


def _segsum(y, nf, gidx3, dst2, zeros):
    ytab = y.reshape(_N * nf, 32)
    return _make_segsum_sc(nf)(ytab, gidx3, dst2, zeros)


def kernel(x, edge_index, batch,
           W1_rel, b1_rel, W1_root, g1, be1,
           W2_rel, b2_rel, W2_root, g2, be2,
           W3_rel, b3_rel, W3_root, g3, be3,
           W4_rel, b4_rel, W4_root, g4, be4):
    src, dst = edge_index[0], edge_index[1]
    r2 = lambda v: v.reshape(1, -1)
    batch3 = batch.reshape(_NB, 1, _B)

    ppw = (_EPAD - _E) // 32
    epw = _E // 32
    lanes = jnp.arange(ppw, dtype=jnp.int32)
    wk = jnp.arange(32, dtype=jnp.int32)
    spread = (wk[:, None] * 8191 + lanes[None, :] * 104729) % _N
    trash = jnp.broadcast_to(_N + (lanes % 16), (32, ppw))
    srcp = jnp.concatenate([src.reshape(32, epw), spread], axis=1).reshape(-1)
    dstp = jnp.concatenate([dst.reshape(32, epw), trash], axis=1).reshape(-1)
    dst2 = dstp.reshape(_NR, 128)
    gidx = {}
    for nf in (1, 2, 4):
        gidx[nf] = (srcp[None, :] * nf
                    + jnp.arange(nf, dtype=jnp.int32)[:, None]).reshape(
                        nf, _NR, 128)
    zeros = jnp.zeros((_NACC, 32), jnp.float32)

    y1 = _linear(x, W1_rel.T)
    a1 = _segsum(y1, 1, gidx[1], dst2, zeros)
    h1 = _make_layer(32, 32, 128, False)(
        a1, x, W1_rel.T, r2(b1_rel), W1_root.T, r2(g1), r2(be1))

    a2 = _segsum(h1, 1, gidx[1], dst2, zeros)
    h2 = _make_layer(32, 64, 32, True)(
        a2, h1, W2_rel.T, r2(b2_rel), W2_root.T, r2(g2), r2(be2))

    a3 = _segsum(h2, 2, gidx[2], dst2, zeros)
    h3 = _make_layer(64, 128, 64, True)(
        a3, h2, W3_rel.T, r2(b3_rel), W3_root.T, r2(g3), r2(be3))

    a4 = _segsum(h3, 4, gidx[4], dst2, zeros)
    return _layer4_pool(a4, h3, W4_rel.T, r2(b4_rel), W4_root.T, r2(g4),
                        r2(be4), batch3)


# speedup vs baseline: 1.0733x; 1.0733x over previous
import functools

import jax
import jax.numpy as jnp
from jax import lax
from jax.experimental import pallas as pl
from jax.experimental.pallas import tpu as pltpu
from jax.experimental.pallas import tpu_sc as plsc

_N = 50000
_E = 800000
_G = 64
_B = 2000
_NB = _N // _B

_RPB = 3
_NBLK = 68
_RPW = _RPB * _NBLK
_NR = 32 * _RPW
_EPAD = 128 * _NR
_TPR = 3128
_NACC = 16 * _TPR


def _ln_relu(t, g, b):
    m = jnp.mean(t, axis=-1, keepdims=True)
    d = t - m
    v = jnp.mean(d * d, axis=-1, keepdims=True)
    y = d * lax.rsqrt(v + 1e-5) * g + b
    return jnp.maximum(y, 0.0)


def _lin_body(x_ref, w_ref, o_ref):
    o_ref[...] = jnp.dot(x_ref[...], w_ref[...],
                         preferred_element_type=jnp.float32)


def _linear(x, wT):
    din, dout = wT.shape
    return pl.pallas_call(
        _lin_body,
        grid=(_NB,),
        in_specs=[
            pl.BlockSpec((_B, din), lambda i: (i, 0)),
            pl.BlockSpec((din, dout), lambda i: (0, 0)),
        ],
        out_specs=pl.BlockSpec((_B, dout), lambda i: (i, 0)),
        out_shape=jax.ShapeDtypeStruct((_N, dout), jnp.float32),
    )(x, wT)


def _make_layer(din, dout, dh, with_wr):
    def body(a_ref, h_ref, wrT_ref, br_ref, wtT_ref, g_ref, b_ref, o_ref):
        a = a_ref[0] + a_ref[1]
        if with_wr:
            t = jnp.dot(a, wrT_ref[...], preferred_element_type=jnp.float32)
        else:
            t = a
        t += br_ref[...]
        t += jnp.dot(h_ref[...], wtT_ref[...],
                     preferred_element_type=jnp.float32)
        o_ref[...] = _ln_relu(t, g_ref[...], b_ref[...])

    def run(aggr, h, wrT, br, wtT, g, b):
        return pl.pallas_call(
            body,
            grid=(_NB,),
            in_specs=[
                pl.BlockSpec((2, _B, din), lambda i: (0, i, 0)),
                pl.BlockSpec((_B, dh), lambda i: (i, 0)),
                pl.BlockSpec(wrT.shape, lambda i: (0, 0)),
                pl.BlockSpec((1, dout), lambda i: (0, 0)),
                pl.BlockSpec((dh, dout), lambda i: (0, 0)),
                pl.BlockSpec((1, dout), lambda i: (0, 0)),
                pl.BlockSpec((1, dout), lambda i: (0, 0)),
            ],
            out_specs=pl.BlockSpec((_B, dout), lambda i: (i, 0)),
            out_shape=jax.ShapeDtypeStruct((_N, dout), jnp.float32),
        )(aggr, h, wrT, br, wtT, g, b)

    return run


def _layer4_pool_body(a_ref, h_ref, wrT_ref, br_ref, wtT_ref, g_ref, b_ref,
                      batch_ref, o_ref, sums, cnt):
    i = pl.program_id(0)

    @pl.when(i == 0)
    def _init():
        sums[...] = jnp.zeros_like(sums)
        cnt[...] = jnp.zeros_like(cnt)

    t = jnp.dot(a_ref[0] + a_ref[1], wrT_ref[...],
                preferred_element_type=jnp.float32)
    t += br_ref[...]
    t += jnp.dot(h_ref[...], wtT_ref[...], preferred_element_type=jnp.float32)
    t = _ln_relu(t, g_ref[...], b_ref[...])

    bb = batch_ref[0]
    mask = (lax.broadcasted_iota(jnp.int32, (_G, _B), 0) == bb).astype(jnp.float32)
    sums[...] += jnp.dot(mask, t, preferred_element_type=jnp.float32)
    cnt[...] += jnp.sum(mask, axis=1, keepdims=True)

    @pl.when(i == _NB - 1)
    def _fin():
        o_ref[...] = sums[...] / jnp.maximum(cnt[:, 0:1], 1.0)


def _layer4_pool(aggr, h, wrT, br, wtT, g, b, batch3):
    din, dout = wrT.shape
    return pl.pallas_call(
        _layer4_pool_body,
        grid=(_NB,),
        in_specs=[
            pl.BlockSpec((2, _B, din), lambda i: (0, i, 0)),
            pl.BlockSpec((_B, din), lambda i: (i, 0)),
            pl.BlockSpec((din, dout), lambda i: (0, 0)),
            pl.BlockSpec((1, dout), lambda i: (0, 0)),
            pl.BlockSpec((din, dout), lambda i: (0, 0)),
            pl.BlockSpec((1, dout), lambda i: (0, 0)),
            pl.BlockSpec((1, dout), lambda i: (0, 0)),
            pl.BlockSpec((1, 1, _B), lambda i: (i, 0, 0)),
        ],
        out_specs=pl.BlockSpec((_G, dout), lambda i: (0, 0)),
        out_shape=jax.ShapeDtypeStruct((_G, dout), jnp.float32),
        scratch_shapes=[
            pltpu.VMEM((_G, dout), jnp.float32),
            pltpu.VMEM((_G, 128), jnp.float32),
        ],
    )(aggr, h, wrT, br, wtT, g, b, batch3)


def _make_segsum_sc(nf):
    mesh = plsc.VectorSubcoreMesh(core_axis_name="c", subcore_axis_name="s")
    BE = _RPB * 128

    @functools.partial(
        pl.kernel, mesh=mesh,
        compiler_params=pltpu.CompilerParams(use_tc_tiling_on_sc=False),
        out_type=jax.ShapeDtypeStruct((2, _N, nf * 32), jnp.float32),
        scratch_types=[
            pltpu.VMEM_SHARED((_NACC, 32), jnp.float32),
            pltpu.VMEM((BE, 32), jnp.float32),
            pltpu.VMEM((BE, 32), jnp.float32),
            pltpu.VMEM((4, _RPB, 128), jnp.int32),
            pltpu.VMEM((4, _RPB, 128), jnp.int32),
            pltpu.SemaphoreType.DMA,
            pltpu.SemaphoreType.DMA,
            pltpu.SemaphoreType.DMA,
            pltpu.SemaphoreType.DMA,
            pltpu.SemaphoreType.DMA,
            pltpu.SemaphoreType.DMA,
            pltpu.SemaphoreType.DMA,
            pltpu.SemaphoreType.DMA,
        ],
    )
    def seg(ytab, gidx3, dst2, zeros, out, acc, buf0, buf1, ig, idx_d,
            semg0, semg1, sems0, sems1, si0, si1, si2, si3):
        c = lax.axis_index("c")
        s = lax.axis_index("s")
        wr0 = (c * 16 + s) * _RPW
        bufs = (buf0, buf1)
        semg = (semg0, semg1)
        sems = (sems0, sems1)
        semi = (si0, si1, si2, si3)

        for fc in range(nf):

            def fire_idx(bb, slot):
                pltpu.async_copy(gidx3.at[fc, pl.ds(wr0 + bb * _RPB, _RPB)],
                                 ig.at[slot], semi[slot])
                pltpu.async_copy(dst2.at[pl.ds(wr0 + bb * _RPB, _RPB)],
                                 idx_d.at[slot], semi[slot])

            def drain_idx(slot):
                pltpu.make_async_copy(gidx3.at[fc, pl.ds(0, _RPB)],
                                      ig.at[slot], semi[slot]).wait()
                pltpu.make_async_copy(dst2.at[pl.ds(0, _RPB)],
                                      idx_d.at[slot], semi[slot]).wait()

            def fire(slot, par):
                for j in range(_RPB):
                    pltpu.async_copy(ytab.at[ig.at[slot, j]],
                                     bufs[par].at[pl.ds(j * 128, 128)],
                                     semg[par])

            def drain_g(par):
                pltpu.make_async_copy(ytab.at[pl.ds(0, BE)], bufs[par],
                                      semg[par]).wait()

            def scat(slot, par):
                for j in range(_RPB):
                    pltpu.async_copy(bufs[par].at[pl.ds(j * 128, 128)],
                                     acc.at[idx_d.at[slot, j]], sems[par],
                                     add=True)

            def drain_s(par):
                pltpu.make_async_copy(ytab.at[pl.ds(0, BE)], bufs[par],
                                      sems[par]).wait()

            pltpu.sync_copy(zeros.at[pl.ds(s * _TPR, _TPR)],
                            acc.at[pl.ds(s * _TPR, _TPR)])
            plsc.subcore_barrier()

            fire_idx(0, 0)
            fire_idx(1, 1)
            fire_idx(2, 2)
            drain_idx(0)
            fire(0, 0)

            def quad(i, carry):
                for q in range(4):
                    p = q % 2
                    if q == 0:
                        @pl.when(i >= 1)
                        def _():
                            drain_s(1)
                    else:
                        drain_s(1 - p)
                    if q == 0:
                        fire_idx(4 * i + 3, 3)
                    else:
                        @pl.when(i <= 15)
                        def _():
                            fire_idx(4 * i + q + 3, (q + 3) % 4)
                    if q == 3:
                        @pl.when(i <= 15)
                        def _():
                            drain_idx(0)
                            fire(0, 1 - p)
                    else:
                        drain_idx(q + 1)
                        fire(q + 1, 1 - p)
                    drain_g(p)
                    scat(q, p)
                return carry

            lax.fori_loop(0, _NBLK // 4, quad, 0)
            drain_s(1)

            plsc.subcore_barrier()
            @pl.when(s <= 14)
            def _wr():
                pltpu.sync_copy(
                    acc.at[pl.ds(s * _TPR, _TPR)],
                    out.at[c, pl.ds(s * _TPR, _TPR), pl.ds(fc * 32, 32)])

            @pl.when(s == 15)
            def _wr_last():
                pltpu.sync_copy(
                    acc.at[pl.ds(15 * _TPR, _N - 15 * _TPR)],
                    out.at[c, pl.ds(15 * _TPR, _N - 15 * _TPR),
                           pl.ds(fc * 32, 32)])

    return seg


def _segsum(y, nf, gidx3, dst2, zeros):
    ytab = y.reshape(_N * nf, 32)
    return _make_segsum_sc(nf)(ytab, gidx3, dst2, zeros)


def kernel(x, edge_index, batch,
           W1_rel, b1_rel, W1_root, g1, be1,
           W2_rel, b2_rel, W2_root, g2, be2,
           W3_rel, b3_rel, W3_root, g3, be3,
           W4_rel, b4_rel, W4_root, g4, be4):
    src, dst = edge_index[0], edge_index[1]
    r2 = lambda v: v.reshape(1, -1)
    batch3 = batch.reshape(_NB, 1, _B)

    ppw = (_EPAD - _E) // 32
    epw = _E // 32
    lanes = jnp.arange(ppw, dtype=jnp.int32)
    wk = jnp.arange(32, dtype=jnp.int32)
    spread = (wk[:, None] * 8191 + lanes[None, :] * 104729) % _N
    trash = jnp.broadcast_to(_N + (lanes % 16), (32, ppw))
    srcp = jnp.concatenate([src.reshape(32, epw), spread], axis=1).reshape(-1)
    dstp = jnp.concatenate([dst.reshape(32, epw), trash], axis=1).reshape(-1)
    dst2 = dstp.reshape(_NR, 128)
    gidx = {}
    for nf in (1, 2, 4):
        gidx[nf] = (srcp[None, :] * nf
                    + jnp.arange(nf, dtype=jnp.int32)[:, None]).reshape(
                        nf, _NR, 128)
    zeros = jnp.zeros((_NACC, 32), jnp.float32)

    y1 = _linear(x, W1_rel.T)
    a1 = _segsum(y1, 1, gidx[1], dst2, zeros)
    h1 = _make_layer(32, 32, 128, False)(
        a1, x, W1_rel.T, r2(b1_rel), W1_root.T, r2(g1), r2(be1))

    a2 = _segsum(h1, 1, gidx[1], dst2, zeros)
    h2 = _make_layer(32, 64, 32, True)(
        a2, h1, W2_rel.T, r2(b2_rel), W2_root.T, r2(g2), r2(be2))

    a3 = _segsum(h2, 2, gidx[2], dst2, zeros)
    h3 = _make_layer(64, 128, 64, True)(
        a3, h2, W3_rel.T, r2(b3_rel), W3_root.T, r2(g3), r2(be3))

    a4 = _segsum(h3, 4, gidx[4], dst2, zeros)
    return _layer4_pool(a4, h3, W4_rel.T, r2(b4_rel), W4_root.T, r2(g4),
                        r2(be4), batch3)


# speedup vs baseline: 1.1095x; 1.0337x over previous
import functools

import jax
import jax.numpy as jnp
from jax import lax
from jax.experimental import pallas as pl
from jax.experimental.pallas import tpu as pltpu
from jax.experimental.pallas import tpu_sc as plsc

_N = 50000
_E = 800000
_G = 64
_B = 5000
_NB = _N // _B

_RPB = 3
_NBLK = 68
_RPW = _RPB * _NBLK
_NR = 32 * _RPW
_EPAD = 128 * _NR
_TPR = 3128
_NACC = 16 * _TPR


def _ln_relu(t, g, b):
    m = jnp.mean(t, axis=-1, keepdims=True)
    d = t - m
    v = jnp.mean(d * d, axis=-1, keepdims=True)
    y = d * lax.rsqrt(v + 1e-5) * g + b
    return jnp.maximum(y, 0.0)


def _lin_body(x_ref, w_ref, o_ref):
    o_ref[...] = jnp.dot(x_ref[...], w_ref[...],
                         preferred_element_type=jnp.float32)


def _linear(x, wT):
    din, dout = wT.shape
    return pl.pallas_call(
        _lin_body,
        grid=(_NB,),
        in_specs=[
            pl.BlockSpec((_B, din), lambda i: (i, 0)),
            pl.BlockSpec((din, dout), lambda i: (0, 0)),
        ],
        out_specs=pl.BlockSpec((_B, dout), lambda i: (i, 0)),
        out_shape=jax.ShapeDtypeStruct((_N, dout), jnp.float32),
    )(x, wT)


def _make_layer(din, dout, dh, with_wr):
    def body(a_ref, h_ref, wrT_ref, br_ref, wtT_ref, g_ref, b_ref, o_ref):
        a = a_ref[0] + a_ref[1]
        if with_wr:
            t = jnp.dot(a, wrT_ref[...], preferred_element_type=jnp.float32)
        else:
            t = a
        t += br_ref[...]
        t += jnp.dot(h_ref[...], wtT_ref[...],
                     preferred_element_type=jnp.float32)
        o_ref[...] = _ln_relu(t, g_ref[...], b_ref[...])

    def run(aggr, h, wrT, br, wtT, g, b):
        return pl.pallas_call(
            body,
            grid=(_NB,),
            in_specs=[
                pl.BlockSpec((2, _B, din), lambda i: (0, i, 0)),
                pl.BlockSpec((_B, dh), lambda i: (i, 0)),
                pl.BlockSpec(wrT.shape, lambda i: (0, 0)),
                pl.BlockSpec((1, dout), lambda i: (0, 0)),
                pl.BlockSpec((dh, dout), lambda i: (0, 0)),
                pl.BlockSpec((1, dout), lambda i: (0, 0)),
                pl.BlockSpec((1, dout), lambda i: (0, 0)),
            ],
            out_specs=pl.BlockSpec((_B, dout), lambda i: (i, 0)),
            out_shape=jax.ShapeDtypeStruct((_N, dout), jnp.float32),
        )(aggr, h, wrT, br, wtT, g, b)

    return run


def _layer4_pool_body(a_ref, h_ref, wrT_ref, br_ref, wtT_ref, g_ref, b_ref,
                      batch_ref, o_ref, sums, cnt):
    i = pl.program_id(0)

    @pl.when(i == 0)
    def _init():
        sums[...] = jnp.zeros_like(sums)
        cnt[...] = jnp.zeros_like(cnt)

    t = jnp.dot(a_ref[0] + a_ref[1], wrT_ref[...],
                preferred_element_type=jnp.float32)
    t += br_ref[...]
    t += jnp.dot(h_ref[...], wtT_ref[...], preferred_element_type=jnp.float32)
    t = _ln_relu(t, g_ref[...], b_ref[...])

    bb = batch_ref[0]
    mask = (lax.broadcasted_iota(jnp.int32, (_G, _B), 0) == bb).astype(jnp.float32)
    sums[...] += jnp.dot(mask, t, preferred_element_type=jnp.float32)
    cnt[...] += jnp.sum(mask, axis=1, keepdims=True)

    @pl.when(i == _NB - 1)
    def _fin():
        o_ref[...] = sums[...] / jnp.maximum(cnt[:, 0:1], 1.0)


def _layer4_pool(aggr, h, wrT, br, wtT, g, b, batch3):
    din, dout = wrT.shape
    return pl.pallas_call(
        _layer4_pool_body,
        grid=(_NB,),
        in_specs=[
            pl.BlockSpec((2, _B, din), lambda i: (0, i, 0)),
            pl.BlockSpec((_B, din), lambda i: (i, 0)),
            pl.BlockSpec((din, dout), lambda i: (0, 0)),
            pl.BlockSpec((1, dout), lambda i: (0, 0)),
            pl.BlockSpec((din, dout), lambda i: (0, 0)),
            pl.BlockSpec((1, dout), lambda i: (0, 0)),
            pl.BlockSpec((1, dout), lambda i: (0, 0)),
            pl.BlockSpec((1, 1, _B), lambda i: (i, 0, 0)),
        ],
        out_specs=pl.BlockSpec((_G, dout), lambda i: (0, 0)),
        out_shape=jax.ShapeDtypeStruct((_G, dout), jnp.float32),
        scratch_shapes=[
            pltpu.VMEM((_G, dout), jnp.float32),
            pltpu.VMEM((_G, 128), jnp.float32),
        ],
    )(aggr, h, wrT, br, wtT, g, b, batch3)


def _make_segsum_sc(nf):
    mesh = plsc.VectorSubcoreMesh(core_axis_name="c", subcore_axis_name="s")
    BE = _RPB * 128

    @functools.partial(
        pl.kernel, mesh=mesh,
        compiler_params=pltpu.CompilerParams(use_tc_tiling_on_sc=False),
        out_type=jax.ShapeDtypeStruct((2, _N, nf * 32), jnp.float32),
        scratch_types=[
            pltpu.VMEM_SHARED((_NACC, 32), jnp.float32),
            pltpu.VMEM((BE, 32), jnp.float32),
            pltpu.VMEM((BE, 32), jnp.float32),
            pltpu.VMEM((4, _RPB, 128), jnp.int32),
            pltpu.VMEM((4, _RPB, 128), jnp.int32),
            pltpu.SemaphoreType.DMA,
            pltpu.SemaphoreType.DMA,
            pltpu.SemaphoreType.DMA,
            pltpu.SemaphoreType.DMA,
            pltpu.SemaphoreType.DMA,
            pltpu.SemaphoreType.DMA,
            pltpu.SemaphoreType.DMA,
            pltpu.SemaphoreType.DMA,
        ],
    )
    def seg(ytab, gidx3, dst2, zeros, out, acc, buf0, buf1, ig, idx_d,
            semg0, semg1, sems0, sems1, si0, si1, si2, si3):
        c = lax.axis_index("c")
        s = lax.axis_index("s")
        wr0 = (c * 16 + s) * _RPW
        bufs = (buf0, buf1)
        semg = (semg0, semg1)
        sems = (sems0, sems1)
        semi = (si0, si1, si2, si3)

        for fc in range(nf):

            def fire_idx(bb, slot):
                pltpu.async_copy(gidx3.at[fc, pl.ds(wr0 + bb * _RPB, _RPB)],
                                 ig.at[slot], semi[slot])
                pltpu.async_copy(dst2.at[pl.ds(wr0 + bb * _RPB, _RPB)],
                                 idx_d.at[slot], semi[slot])

            def drain_idx(slot):
                pltpu.make_async_copy(gidx3.at[fc, pl.ds(0, _RPB)],
                                      ig.at[slot], semi[slot]).wait()
                pltpu.make_async_copy(dst2.at[pl.ds(0, _RPB)],
                                      idx_d.at[slot], semi[slot]).wait()

            def fire(slot, par):
                for j in range(_RPB):
                    pltpu.async_copy(ytab.at[ig.at[slot, j]],
                                     bufs[par].at[pl.ds(j * 128, 128)],
                                     semg[par])

            def drain_g(par):
                pltpu.make_async_copy(ytab.at[pl.ds(0, BE)], bufs[par],
                                      semg[par]).wait()

            def scat(slot, par):
                for j in range(_RPB):
                    pltpu.async_copy(bufs[par].at[pl.ds(j * 128, 128)],
                                     acc.at[idx_d.at[slot, j]], sems[par],
                                     add=True)

            def drain_s(par):
                pltpu.make_async_copy(ytab.at[pl.ds(0, BE)], bufs[par],
                                      sems[par]).wait()

            pltpu.sync_copy(zeros.at[pl.ds(s * _TPR, _TPR)],
                            acc.at[pl.ds(s * _TPR, _TPR)])
            plsc.subcore_barrier()

            fire_idx(0, 0)
            fire_idx(1, 1)
            fire_idx(2, 2)
            drain_idx(0)
            fire(0, 0)

            def quad(i, carry):
                for q in range(4):
                    p = q % 2
                    if q == 0:
                        @pl.when(i >= 1)
                        def _():
                            drain_s(1)
                    else:
                        drain_s(1 - p)
                    if q == 0:
                        fire_idx(4 * i + 3, 3)
                    else:
                        @pl.when(i <= 15)
                        def _():
                            fire_idx(4 * i + q + 3, (q + 3) % 4)
                    if q == 3:
                        @pl.when(i <= 15)
                        def _():
                            drain_idx(0)
                            fire(0, 1 - p)
                    else:
                        drain_idx(q + 1)
                        fire(q + 1, 1 - p)
                    drain_g(p)
                    scat(q, p)
                return carry

            lax.fori_loop(0, _NBLK // 4, quad, 0)
            drain_s(1)

            plsc.subcore_barrier()
            @pl.when(s <= 14)
            def _wr():
                pltpu.sync_copy(
                    acc.at[pl.ds(s * _TPR, _TPR)],
                    out.at[c, pl.ds(s * _TPR, _TPR), pl.ds(fc * 32, 32)])

            @pl.when(s == 15)
            def _wr_last():
                pltpu.sync_copy(
                    acc.at[pl.ds(15 * _TPR, _N - 15 * _TPR)],
                    out.at[c, pl.ds(15 * _TPR, _N - 15 * _TPR),
                           pl.ds(fc * 32, 32)])

    return seg


def _segsum(y, nf, gidx3, dst2, zeros):
    ytab = y.reshape(_N * nf, 32)
    return _make_segsum_sc(nf)(ytab, gidx3, dst2, zeros)


def kernel(x, edge_index, batch,
           W1_rel, b1_rel, W1_root, g1, be1,
           W2_rel, b2_rel, W2_root, g2, be2,
           W3_rel, b3_rel, W3_root, g3, be3,
           W4_rel, b4_rel, W4_root, g4, be4):
    src, dst = edge_index[0], edge_index[1]
    r2 = lambda v: v.reshape(1, -1)
    batch3 = batch.reshape(_NB, 1, _B)

    ppw = (_EPAD - _E) // 32
    epw = _E // 32
    lanes = jnp.arange(ppw, dtype=jnp.int32)
    wk = jnp.arange(32, dtype=jnp.int32)
    spread = (wk[:, None] * 8191 + lanes[None, :] * 104729) % _N
    trash = jnp.broadcast_to(_N + (lanes % 16), (32, ppw))
    srcp = jnp.concatenate([src.reshape(32, epw), spread], axis=1).reshape(-1)
    dstp = jnp.concatenate([dst.reshape(32, epw), trash], axis=1).reshape(-1)
    dst2 = dstp.reshape(_NR, 128)
    gidx = {}
    for nf in (1, 2, 4):
        gidx[nf] = (srcp[None, :] * nf
                    + jnp.arange(nf, dtype=jnp.int32)[:, None]).reshape(
                        nf, _NR, 128)
    zeros = jnp.zeros((_NACC, 32), jnp.float32)

    y1 = _linear(x, W1_rel.T)
    a1 = _segsum(y1, 1, gidx[1], dst2, zeros)
    h1 = _make_layer(32, 32, 128, False)(
        a1, x, W1_rel.T, r2(b1_rel), W1_root.T, r2(g1), r2(be1))

    a2 = _segsum(h1, 1, gidx[1], dst2, zeros)
    h2 = _make_layer(32, 64, 32, True)(
        a2, h1, W2_rel.T, r2(b2_rel), W2_root.T, r2(g2), r2(be2))

    a3 = _segsum(h2, 2, gidx[2], dst2, zeros)
    h3 = _make_layer(64, 128, 64, True)(
        a3, h2, W3_rel.T, r2(b3_rel), W3_root.T, r2(g3), r2(be3))

    a4 = _segsum(h3, 4, gidx[4], dst2, zeros)
    return _layer4_pool(a4, h3, W4_rel.T, r2(b4_rel), W4_root.T, r2(g4),
                        r2(be4), batch3)
